# Initial kernel scaffold; baseline (speedup 1.0000x reference)
import jax
import jax.numpy as jnp
from jax.experimental import pallas as pl


def kernel(x, w1, w3, w2):
    raise NotImplementedError("write your pallas kernel here")


# speedup vs baseline: 6.0535x; 6.0535x over previous
import jax
import jax.numpy as jnp
from jax.experimental import pallas as pl
from jax.experimental.pallas import tpu as pltpu


def _mlp_kernel(x_ref, w1_ref, w3_ref, w2_ref, y_ref):
    j = pl.program_id(1)
    x = x_ref[...]
    w1 = w1_ref[...].astype(jnp.bfloat16)
    w3 = w3_ref[...].astype(jnp.bfloat16)
    g = jnp.dot(x, w1, preferred_element_type=jnp.float32)
    u = jnp.dot(x, w3, preferred_element_type=jnp.float32)
    h = (g * (1.0 / (1.0 + jnp.exp(-g))) * u).astype(jnp.bfloat16)
    w2 = w2_ref[...].astype(jnp.bfloat16)
    contrib = jnp.dot(h, w2, preferred_element_type=jnp.float32)

    @pl.when(j == 0)
    def _():
        y_ref[...] = contrib

    @pl.when(j != 0)
    def _():
        y_ref[...] += contrib


def kernel(x, w1, w3, w2):
    B, S, H = x.shape
    I = w1.shape[1]
    M = B * S
    x2d = x.reshape(M, H).astype(jnp.bfloat16)

    bm = 512 if M % 512 == 0 else M
    bi = 256 if I % 256 == 0 else I

    y = pl.pallas_call(
        _mlp_kernel,
        out_shape=jax.ShapeDtypeStruct((M, H), jnp.float32),
        grid=(M // bm, I // bi),
        in_specs=[
            pl.BlockSpec((bm, H), lambda i, j: (i, 0)),
            pl.BlockSpec((H, bi), lambda i, j: (0, j)),
            pl.BlockSpec((H, bi), lambda i, j: (0, j)),
            pl.BlockSpec((bi, H), lambda i, j: (j, 0)),
        ],
        out_specs=pl.BlockSpec((bm, H), lambda i, j: (i, 0)),
        compiler_params=pltpu.CompilerParams(
            dimension_semantics=("parallel", "arbitrary"),
            vmem_limit_bytes=64 * 1024 * 1024,
        ),
    )(x2d, w1, w3, w2)
    return y.reshape(B, S, H)


# speedup vs baseline: 7.8871x; 1.3029x over previous
import jax
import jax.numpy as jnp
from jax.experimental import pallas as pl
from jax.experimental.pallas import tpu as pltpu

_H_CHUNK = 1024


def _mlp_kernel(x_hbm, w1_ref, w3_ref, w2_ref, y_hbm,
                x_vmem, acc_ref, in_sem, out_sem):
    i = pl.program_id(0)
    j = pl.program_id(1)
    nj = pl.num_programs(1)
    bm, H = acc_ref.shape

    @pl.when(j == 0)
    def _():
        pltpu.make_async_copy(
            x_hbm.at[pl.ds(i * bm, bm), :], x_vmem, in_sem).start()
        pltpu.make_async_copy(
            x_hbm.at[pl.ds(i * bm, bm), :], x_vmem, in_sem).wait()
        acc_ref[...] = jnp.zeros_like(acc_ref)

    x = x_vmem[...]
    w1 = w1_ref[...].astype(jnp.bfloat16)
    w3 = w3_ref[...].astype(jnp.bfloat16)
    g = jnp.dot(x, w1, preferred_element_type=jnp.float32)
    u = jnp.dot(x, w3, preferred_element_type=jnp.float32)
    h = (g * (1.0 / (1.0 + jnp.exp(-g))) * u).astype(jnp.bfloat16)
    for c in range(0, H, _H_CHUNK):
        w2c = w2_ref[:, c:c + _H_CHUNK].astype(jnp.bfloat16)
        acc_ref[:, c:c + _H_CHUNK] += jnp.dot(
            h, w2c, preferred_element_type=jnp.float32)

    @pl.when(j == nj - 1)
    def _():
        pltpu.make_async_copy(
            acc_ref, y_hbm.at[pl.ds(i * bm, bm), :], out_sem).start()
        pltpu.make_async_copy(
            acc_ref, y_hbm.at[pl.ds(i * bm, bm), :], out_sem).wait()


def kernel(x, w1, w3, w2):
    B, S, H = x.shape
    I = w1.shape[1]
    M = B * S
    x2d = x.reshape(M, H).astype(jnp.bfloat16)

    bm = 1024 if M % 1024 == 0 else M
    bi = 256 if I % 256 == 0 else I

    y = pl.pallas_call(
        _mlp_kernel,
        out_shape=jax.ShapeDtypeStruct((M, H), jnp.float32),
        grid=(M // bm, I // bi),
        in_specs=[
            pl.BlockSpec(memory_space=pl.ANY),
            pl.BlockSpec((H, bi), lambda i, j: (0, j)),
            pl.BlockSpec((H, bi), lambda i, j: (0, j)),
            pl.BlockSpec((bi, H), lambda i, j: (j, 0)),
        ],
        out_specs=pl.BlockSpec(memory_space=pl.ANY),
        scratch_shapes=[
            pltpu.VMEM((bm, H), jnp.bfloat16),
            pltpu.VMEM((bm, H), jnp.float32),
            pltpu.SemaphoreType.DMA,
            pltpu.SemaphoreType.DMA,
        ],
        compiler_params=pltpu.CompilerParams(
            dimension_semantics=("parallel", "arbitrary"),
            vmem_limit_bytes=64 * 1024 * 1024,
        ),
    )(x2d, w1, w3, w2)
    return y.reshape(B, S, H)


# speedup vs baseline: 7.9819x; 1.0120x over previous
import jax
import jax.numpy as jnp
from jax.experimental import pallas as pl
from jax.experimental.pallas import tpu as pltpu

_H_CHUNK = 1024
_NSLOT = 2


def _mlp_kernel(x_hbm, w1_hbm, w3_hbm, w2_hbm, y_hbm,
                x_vmem, acc_ref, w1_buf, w3_buf, w2_buf,
                x_sem, w_sems, out_sem):
    i = pl.program_id(0)
    j = pl.program_id(1)
    nj = pl.num_programs(1)
    bm, H = acc_ref.shape
    bi = w2_buf.shape[1]

    def start_fetch(jj, slot):
        pltpu.make_async_copy(
            w1_hbm.at[:, pl.ds(jj * bi, bi)], w1_buf.at[slot],
            w_sems.at[0, slot]).start()
        pltpu.make_async_copy(
            w3_hbm.at[:, pl.ds(jj * bi, bi)], w3_buf.at[slot],
            w_sems.at[1, slot]).start()
        pltpu.make_async_copy(
            w2_hbm.at[pl.ds(jj * bi, bi), :], w2_buf.at[slot],
            w_sems.at[2, slot]).start()

    def wait_fetch(slot):
        pltpu.make_async_copy(
            w1_buf.at[slot], w1_buf.at[slot], w_sems.at[0, slot]).wait()
        pltpu.make_async_copy(
            w3_buf.at[slot], w3_buf.at[slot], w_sems.at[1, slot]).wait()
        pltpu.make_async_copy(
            w2_buf.at[slot], w2_buf.at[slot], w_sems.at[2, slot]).wait()

    slot = jax.lax.rem(j, _NSLOT)
    next_slot = jax.lax.rem(j + 1, _NSLOT)

    @pl.when(j == 0)
    def _():
        pltpu.make_async_copy(
            x_hbm.at[pl.ds(i * bm, bm), :], x_vmem, x_sem).start()
        start_fetch(0, 0)
        acc_ref[...] = jnp.zeros_like(acc_ref)

    @pl.when(j + 1 < nj)
    def _():
        start_fetch(j + 1, next_slot)

    @pl.when(j == 0)
    def _():
        pltpu.make_async_copy(x_vmem, x_vmem, x_sem).wait()

    wait_fetch(slot)

    x = x_vmem[...]
    w1 = w1_buf[slot].astype(jnp.bfloat16)
    w3 = w3_buf[slot].astype(jnp.bfloat16)
    g = jnp.dot(x, w1, preferred_element_type=jnp.float32)
    u = jnp.dot(x, w3, preferred_element_type=jnp.float32)
    h = (g * (1.0 / (1.0 + jnp.exp(-g))) * u).astype(jnp.bfloat16)
    for c in range(0, H, _H_CHUNK):
        w2c = w2_buf[slot, :, c:c + _H_CHUNK].astype(jnp.bfloat16)
        acc_ref[:, c:c + _H_CHUNK] += jnp.dot(
            h, w2c, preferred_element_type=jnp.float32)

    @pl.when(j == nj - 1)
    def _():
        pltpu.make_async_copy(
            acc_ref, y_hbm.at[pl.ds(i * bm, bm), :], out_sem).start()
        pltpu.make_async_copy(
            acc_ref, y_hbm.at[pl.ds(i * bm, bm), :], out_sem).wait()


def kernel(x, w1, w3, w2):
    B, S, H = x.shape
    I = w1.shape[1]
    M = B * S
    x2d = x.reshape(M, H).astype(jnp.bfloat16)

    bm = 1024 if M % 1024 == 0 else M
    bi = 256 if I % 256 == 0 else I

    y = pl.pallas_call(
        _mlp_kernel,
        out_shape=jax.ShapeDtypeStruct((M, H), jnp.float32),
        grid=(M // bm, I // bi),
        in_specs=[
            pl.BlockSpec(memory_space=pl.ANY),
            pl.BlockSpec(memory_space=pl.ANY),
            pl.BlockSpec(memory_space=pl.ANY),
            pl.BlockSpec(memory_space=pl.ANY),
        ],
        out_specs=pl.BlockSpec(memory_space=pl.ANY),
        scratch_shapes=[
            pltpu.VMEM((bm, H), jnp.bfloat16),
            pltpu.VMEM((bm, H), jnp.float32),
            pltpu.VMEM((_NSLOT, H, bi), jnp.float32),
            pltpu.VMEM((_NSLOT, H, bi), jnp.float32),
            pltpu.VMEM((_NSLOT, bi, H), jnp.float32),
            pltpu.SemaphoreType.DMA,
            pltpu.SemaphoreType.DMA((3, _NSLOT)),
            pltpu.SemaphoreType.DMA,
        ],
        compiler_params=pltpu.CompilerParams(
            dimension_semantics=("parallel", "arbitrary"),
            vmem_limit_bytes=64 * 1024 * 1024,
        ),
    )(x2d, w1, w3, w2)
    return y.reshape(B, S, H)
